# Initial kernel scaffold; baseline (speedup 1.0000x reference)
import jax
import jax.numpy as jnp
from jax.experimental import pallas as pl


def kernel(hidden_states, Wq, Wk, Wv, Wb, Wa, Wg_router, A_log, dt_bias, Wg_gate, o_norm_w, Wo):
    raise NotImplementedError("write your pallas kernel here")


# speedup vs baseline: 5.4780x; 5.4780x over previous
import functools

import jax
import jax.numpy as jnp
import numpy as np
from jax.experimental import pallas as pl
from jax.experimental.pallas import tpu as pltpu

B, S, HID = 1, 2048, 1024
NH, DK = 4, 64
DV = 64
KD = NH * DK
VD = NH * DV
M, TOPK = 8, 2
EPS = 1e-5
C = 64
NC = S // C
P = M * NH
TS = 256
SCALE = DK ** -0.5


def _silu(x):
    return x * jax.nn.sigmoid(x)


def _proj_kernel(hs_ref, wbig_ref, wsmall_ref, ad_ref,
                 q_ref, k_ref, v_ref, gate_ref, scal_ref):
    x = hs_ref[...]
    z = jnp.dot(x, wbig_ref[...], preferred_element_type=jnp.float32)
    zs = jnp.dot(x, wsmall_ref[...], preferred_element_type=jnp.float32)

    for h in range(NH):
        sq = _silu(z[:, h * DK:(h + 1) * DK])
        q_ref[h] = sq * jax.lax.rsqrt(
            jnp.sum(sq * sq, axis=1, keepdims=True) + 1e-6)
        sk = _silu(z[:, KD + h * DK:KD + (h + 1) * DK])
        k_ref[h] = sk * jax.lax.rsqrt(
            jnp.sum(sk * sk, axis=1, keepdims=True) + 1e-6)
        v_ref[h] = _silu(z[:, 2 * KD + h * DV:2 * KD + (h + 1) * DV])
    gate_ref[...] = _silu(z[:, 3 * KD:3 * KD + VD])

    beta = jax.nn.sigmoid(zs[:, 0:NH])
    az = zs[:, NH:2 * NH] + ad_ref[0:1, NH:2 * NH]
    sp = jnp.maximum(az, 0.0) + jnp.log(1.0 + jnp.exp(-jnp.abs(az)))
    g = -jnp.exp(ad_ref[0:1, 0:NH]) * sp

    r = zs[:, 2 * NH:2 * NH + M]
    r = r - jnp.max(r, axis=1, keepdims=True)
    e = jnp.exp(r)
    s = e / jnp.sum(e, axis=1, keepdims=True)
    iota = jax.lax.broadcasted_iota(jnp.int32, (TS, M), 1)
    m1 = jnp.max(s, axis=1, keepdims=True)
    i1 = jnp.min(jnp.where(s == m1, iota, M), axis=1, keepdims=True)
    s2 = jnp.where(iota == i1, -1.0, s)
    m2 = jnp.max(s2, axis=1, keepdims=True)
    i2 = jnp.min(jnp.where(s2 == m2, iota, M), axis=1, keepdims=True)
    tot = m1 + m2
    rw = jnp.where(iota == i1, m1 / tot, 0.0) + \
        jnp.where(iota == i2, m2 / tot, 0.0)
    maskf = (rw > 0).astype(jnp.float32)

    b_all = jnp.concatenate(
        [beta[:, h:h + 1] * maskf for h in range(NH)], axis=1)
    g_all = jnp.concatenate(
        [g[:, h:h + 1] * maskf for h in range(NH)], axis=1)
    rw_rep = jnp.concatenate([rw] * NH, axis=1)
    scal_ref[...] = jnp.concatenate(
        [b_all, g_all, rw_rep, jnp.zeros((TS, 32), jnp.float32)], axis=1)


def _scan_kernel(k_ref, q_ref, v_ref, sc_ref, o_ref,
                 state_ref, gram_ref, qk_ref):
    n = pl.program_id(0)
    p = pl.program_id(1)
    mloc = jax.lax.rem(p, M)

    kk = k_ref[0]
    qq = q_ref[0]
    vv = v_ref[0]
    b_col = sc_ref[0, :, 0:1]
    g_col = sc_ref[0, :, 1:2]
    rw_col = sc_ref[0, :, 2:3]

    @pl.when(n == 0)
    def _init_state():
        state_ref[p] = jnp.zeros((DK, DV), jnp.float32)

    @pl.when(mloc == 0)
    def _init_grams():
        gram_ref[...] = jnp.dot(kk, kk.T, preferred_element_type=jnp.float32)
        qk_ref[...] = jnp.dot(qq, kk.T, preferred_element_type=jnp.float32)

    rows = jax.lax.broadcasted_iota(jnp.int32, (C, C), 0)
    cols = jax.lax.broadcasted_iota(jnp.int32, (C, C), 1)
    lower = rows > cols
    lowereq = rows >= cols
    ltri = lowereq.astype(jnp.float32)

    G_col = jnp.dot(ltri, g_col, preferred_element_type=jnp.float32)
    A_col = jnp.exp(G_col)
    Gmat = jnp.broadcast_to(G_col, (C, C))
    Ediff = Gmat - Gmat.T
    El = jnp.where(lower, jnp.exp(jnp.where(lower, Ediff, 0.0)), 0.0)
    Ele = jnp.where(lowereq, jnp.exp(jnp.where(lowereq, Ediff, 0.0)), 0.0)

    eye = jnp.where(rows == cols, 1.0, 0.0)
    Nmat = -(b_col * El * gram_ref[...])
    X = eye + Nmat
    Pm = Nmat
    for _ in range(5):
        Pm = jnp.dot(Pm, Pm, preferred_element_type=jnp.float32)
        X = X + jnp.dot(X, Pm, preferred_element_type=jnp.float32)

    S0 = state_ref[p]
    W1 = jnp.dot(X, b_col * vv, preferred_element_type=jnp.float32)
    W2 = jnp.dot(X, (b_col * A_col) * kk, preferred_element_type=jnp.float32)
    U = W1 - jnp.dot(W2, S0, preferred_element_type=jnp.float32)

    attn = (SCALE * rw_col) * Ele * qk_ref[...]
    qA = (SCALE * rw_col * A_col) * qq
    contrib = jnp.dot(qA, S0, preferred_element_type=jnp.float32) + \
        jnp.dot(attn, U, preferred_element_type=jnp.float32)

    aC = A_col[C - 1:C, 0:1]
    kbar = jnp.exp(G_col[C - 1:C, 0:1] - G_col) * kk
    state_ref[p] = aC * S0 + jax.lax.dot_general(
        kbar, U, (((0,), (0,)), ((), ())),
        preferred_element_type=jnp.float32)

    @pl.when(mloc == 0)
    def _first():
        o_ref[0] = contrib

    @pl.when(mloc != 0)
    def _accum():
        o_ref[0] = o_ref[0] + contrib


def _out_kernel(o_ref, gate_ref, onw_ref, wot_ref, out_ref):
    ys = []
    for h in range(NH):
        oh = o_ref[h]
        ms = jnp.sum(oh * oh, axis=1, keepdims=True) * (1.0 / DV)
        rms = oh * jax.lax.rsqrt(ms + EPS)
        ys.append(rms * onw_ref[0:1, :] * gate_ref[:, h * DV:(h + 1) * DV])
    y = jnp.concatenate(ys, axis=1)
    out_ref[...] = jnp.dot(y, wot_ref[...], preferred_element_type=jnp.float32)


@jax.jit
def kernel(hidden_states, Wq, Wk, Wv, Wb, Wa, Wg_router, A_log, dt_bias,
           Wg_gate, o_norm_w, Wo):
    hs = hidden_states.reshape(S, HID)
    wbig = jnp.concatenate([Wq, Wk, Wv, Wg_gate], axis=0).T
    wsmall = jnp.concatenate([Wb, Wa, Wg_router], axis=0).T
    ad = jnp.concatenate([A_log, dt_bias]).reshape(1, 2 * NH)
    ad = jnp.pad(ad, ((0, 0), (0, M)))

    q, k, v, gate, scal = pl.pallas_call(
        _proj_kernel,
        grid=(S // TS,),
        in_specs=[
            pl.BlockSpec((TS, HID), lambda t: (t, 0)),
            pl.BlockSpec((HID, 3 * KD + VD), lambda t: (0, 0)),
            pl.BlockSpec((HID, 16), lambda t: (0, 0)),
            pl.BlockSpec((1, 16), lambda t: (0, 0)),
        ],
        out_specs=[
            pl.BlockSpec((NH, TS, DK), lambda t: (0, t, 0)),
            pl.BlockSpec((NH, TS, DK), lambda t: (0, t, 0)),
            pl.BlockSpec((NH, TS, DV), lambda t: (0, t, 0)),
            pl.BlockSpec((TS, VD), lambda t: (t, 0)),
            pl.BlockSpec((TS, 128), lambda t: (t, 0)),
        ],
        out_shape=[
            jax.ShapeDtypeStruct((NH, S, DK), jnp.float32),
            jax.ShapeDtypeStruct((NH, S, DK), jnp.float32),
            jax.ShapeDtypeStruct((NH, S, DV), jnp.float32),
            jax.ShapeDtypeStruct((S, VD), jnp.float32),
            jax.ShapeDtypeStruct((S, 128), jnp.float32),
        ],
    )(hs, wbig, wsmall, ad)

    scal_pair = jnp.stack(
        [scal[:, 0:P].T, scal[:, P:2 * P].T, scal[:, 2 * P:3 * P].T],
        axis=-1)
    scal_pair = jnp.pad(scal_pair, ((0, 0), (0, 0), (0, 5)))

    o = pl.pallas_call(
        _scan_kernel,
        grid=(NC, P),
        in_specs=[
            pl.BlockSpec((1, C, DK), lambda n, p: (p // M, n, 0)),
            pl.BlockSpec((1, C, DK), lambda n, p: (p // M, n, 0)),
            pl.BlockSpec((1, C, DV), lambda n, p: (p // M, n, 0)),
            pl.BlockSpec((1, C, 8), lambda n, p: (p, n, 0)),
        ],
        out_specs=pl.BlockSpec((1, C, DV), lambda n, p: (p // M, n, 0)),
        out_shape=jax.ShapeDtypeStruct((NH, S, DV), jnp.float32),
        scratch_shapes=[
            pltpu.VMEM((P, DK, DV), jnp.float32),
            pltpu.VMEM((C, C), jnp.float32),
            pltpu.VMEM((C, C), jnp.float32),
        ],
        compiler_params=pltpu.CompilerParams(
            dimension_semantics=("arbitrary", "arbitrary")),
    )(k, q, v, scal_pair)

    out = pl.pallas_call(
        _out_kernel,
        grid=(S // TS,),
        in_specs=[
            pl.BlockSpec((NH, TS, DV), lambda t: (0, t, 0)),
            pl.BlockSpec((TS, VD), lambda t: (t, 0)),
            pl.BlockSpec((1, DV), lambda t: (0, 0)),
            pl.BlockSpec((VD, HID), lambda t: (0, 0)),
        ],
        out_specs=pl.BlockSpec((TS, HID), lambda t: (t, 0)),
        out_shape=jax.ShapeDtypeStruct((S, HID), jnp.float32),
    )(o, gate, o_norm_w.reshape(1, DV), Wo.T)

    return out.reshape(B, S, HID)


# speedup vs baseline: 13.0630x; 2.3846x over previous
import functools

import jax
import jax.numpy as jnp
import numpy as np
from jax.experimental import pallas as pl
from jax.experimental.pallas import tpu as pltpu
from jax.experimental.pallas import tpu_sc as plsc

B, S, HID = 1, 2048, 1024
NH, DK = 4, 64
DV = 64
KD = NH * DK
VD = NH * DV
M, TOPK = 8, 2
EPS = 1e-5
C = 64
NC = S // C
P = M * NH
TS = 256
SCALE = DK ** -0.5


def _silu(x):
    return x * jax.nn.sigmoid(x)


def _logits_kernel(hs_ref, wr_ref, out_ref):
    out_ref[...] = jax.lax.dot_general(
        wr_ref[...], hs_ref[...], (((1,), (1,)), ((), ())),
        preferred_element_type=jnp.float32)


_TPW = 128


def _router_sc(logits_hbm, out_hbm, buf, obuf):
    wid = jax.lax.axis_index("s") * 2 + jax.lax.axis_index("c")
    base = wid * _TPW

    @pl.when(wid < S // _TPW)
    def _work():
        _router_body(logits_hbm, out_hbm, buf, obuf, base)


def _router_body(logits_hbm, out_hbm, buf, obuf, base):
    pltpu.sync_copy(logits_hbm.at[:, pl.ds(base, _TPW)], buf)
    for t in range(_TPW // 16):
        sl = [buf[m, pl.ds(t * 16, 16)] for m in range(M)]
        mx = sl[0]
        for m in range(1, M):
            mx = jnp.maximum(mx, sl[m])
        es = [jnp.exp(x - mx) for x in sl]
        tot = es[0]
        for m in range(1, M):
            tot = tot + es[m]
        ss = [e / tot for e in es]
        m1 = ss[0]
        for m in range(1, M):
            m1 = jnp.maximum(m1, ss[m])
        i1 = jnp.full((16,), 0, jnp.int32)
        for m in range(M - 1, -1, -1):
            i1 = jnp.where(ss[m] == m1, m, i1)
        s2 = [jnp.where(i1 == m, -1.0, ss[m]) for m in range(M)]
        m2 = s2[0]
        for m in range(1, M):
            m2 = jnp.maximum(m2, s2[m])
        i2 = jnp.full((16,), 0, jnp.int32)
        for m in range(M - 1, -1, -1):
            i2 = jnp.where(s2[m] == m2, m, i2)
        w1 = m1 / (m1 + m2)
        w2 = m2 / (m1 + m2)
        for m in range(M):
            obuf[m, pl.ds(t * 16, 16)] = \
                jnp.where(i1 == m, w1, 0.0) + jnp.where(i2 == m, w2, 0.0)
    pltpu.sync_copy(obuf, out_hbm.at[:, pl.ds(base, _TPW)])


def _proj_kernel(hs_ref, wbig_ref, wsmall_ref, adT_ref,
                 q_ref, k_ref, v_ref, gate_ref, bg_ref):
    x = hs_ref[...]
    z = jnp.dot(x, wbig_ref[...], preferred_element_type=jnp.float32)
    zsT = jax.lax.dot_general(
        wsmall_ref[...], x, (((1,), (1,)), ((), ())),
        preferred_element_type=jnp.float32)

    for h in range(NH):
        sq = _silu(z[:, h * DK:(h + 1) * DK])
        q_ref[h] = sq * jax.lax.rsqrt(
            jnp.sum(sq * sq, axis=1, keepdims=True) + 1e-6)
        sk = _silu(z[:, KD + h * DK:KD + (h + 1) * DK])
        k_ref[h] = sk * jax.lax.rsqrt(
            jnp.sum(sk * sk, axis=1, keepdims=True) + 1e-6)
        v_ref[h] = _silu(z[:, 2 * KD + h * DV:2 * KD + (h + 1) * DV])
    gate_ref[...] = _silu(z[:, 3 * KD:3 * KD + VD])

    bg_ref[0:NH] = jax.nn.sigmoid(zsT[0:NH])
    az = zsT[NH:2 * NH] + adT_ref[NH:2 * NH, 0:1]
    sp = jnp.maximum(az, 0.0) + jnp.log(1.0 + jnp.exp(-jnp.abs(az)))
    bg_ref[NH:2 * NH] = -jnp.exp(adT_ref[0:NH, 0:1]) * sp


def _scan_kernel(k_ref, q_ref, v_ref, bg_ref, rw_ref, o_ref, state_ref):
    n = pl.program_id(0)

    @pl.when(n == 0)
    def _init_state():
        state_ref[...] = jnp.zeros((NH, M * DK, DV), jnp.float32)

    MC = M * C
    rows = jax.lax.broadcasted_iota(jnp.int32, (C, C), 0)
    cols = jax.lax.broadcasted_iota(jnp.int32, (C, C), 1)
    ltri = (rows >= cols).astype(jnp.float32)
    eye = jnp.where(rows == cols, 1.0, 0.0)
    rowsb = jax.lax.broadcasted_iota(jnp.int32, (C, MC), 0)
    colsb = jax.lax.broadcasted_iota(jnp.int32, (C, MC), 1)
    cmod = jax.lax.rem(colsb, C)
    lower_b = rowsb > cmod
    lowereq_b = rowsb >= cmod
    eye_b = (rowsb == cmod).astype(jnp.float32)
    E = (jax.lax.broadcasted_iota(jnp.int32, (M, MC), 1) // C ==
         jax.lax.broadcasted_iota(jnp.int32, (M, MC), 0)).astype(jnp.float32)

    for c in range(2):
        bg_cols = jnp.transpose(bg_ref[c])
        rw_cols_all = jnp.transpose(rw_ref[c])
        mask_cols = (rw_cols_all > 0).astype(jnp.float32)
        for h in range(NH):
            b_cols = bg_cols[:, h:h + 1] * mask_cols
            g_cols = bg_cols[:, NH + h:NH + h + 1] * mask_cols
            _scan_head(h, c, k_ref, q_ref, v_ref, b_cols, g_cols,
                       rw_cols_all, o_ref, state_ref, ltri, eye,
                       lower_b, lowereq_b, eye_b, E)


def _scan_head(h, c, k_ref, q_ref, v_ref, b_cols, g_cols, rw_cols,
               o_ref, state_ref, ltri, eye, lower_b, lowereq_b, eye_b, E):
    kk = k_ref[h, c * C:(c + 1) * C]
    qq = q_ref[h, c * C:(c + 1) * C]
    vv = v_ref[h, c * C:(c + 1) * C]
    gram = jnp.dot(kk, kk.T, preferred_element_type=jnp.float32)
    qk = jnp.dot(qq, kk.T, preferred_element_type=jnp.float32)

    G_cols = jnp.dot(ltri, g_cols, preferred_element_type=jnp.float32)
    A_cols = jnp.exp(G_cols)

    Gbig = jnp.dot(G_cols, E, preferred_element_type=jnp.float32)
    bbig = jnp.dot(b_cols, E, preferred_element_type=jnp.float32)
    rwsbig = jnp.dot(SCALE * rw_cols, E, preferred_element_type=jnp.float32)
    rwAbig = jnp.dot(SCALE * rw_cols * A_cols, E,
                     preferred_element_type=jnp.float32)
    G_rowflat = jnp.sum(Gbig * eye_b, axis=0, keepdims=True)
    b_rowflat = jnp.sum(bbig * eye_b, axis=0, keepdims=True)
    bA_rowflat = b_rowflat * jnp.exp(G_rowflat)
    Ediff = Gbig - G_rowflat
    El = jnp.where(lower_b, jnp.exp(jnp.where(lower_b, Ediff, 0.0)), 0.0)
    Ele = jnp.where(lowereq_b, jnp.exp(jnp.where(lowereq_b, Ediff, 0.0)), 0.0)
    gramt = jnp.concatenate([gram] * M, axis=1)
    qkt = jnp.concatenate([qk] * M, axis=1)
    Nall = -(bbig * El * gramt)
    attn_all = rwsbig * Ele * qkt
    qA_all = rwAbig * jnp.concatenate([qq] * M, axis=1)

    S0_all = state_ref[h]

    Xb_l, XbA_l = [], []
    for m in range(M):
        Nmat = Nall[:, m * C:(m + 1) * C]
        X = eye + Nmat
        Q = jnp.dot(Nmat, Nmat, preferred_element_type=jnp.float32)
        for _ in range(4):
            R = jnp.dot(jnp.concatenate([Q, X], axis=0), Q,
                        preferred_element_type=jnp.float32)
            X = X + R[C:]
            Q = R[:C]
        X = X + jnp.dot(X, Q, preferred_element_type=jnp.float32)
        Xb_l.append(X * b_rowflat[:, m * C:(m + 1) * C])
        XbA_l.append(X * bA_rowflat[:, m * C:(m + 1) * C])

    Xb_all = jnp.concatenate(Xb_l, axis=0)
    XbA_all = jnp.concatenate(XbA_l, axis=0)
    W1_all = jnp.dot(Xb_all, vv, preferred_element_type=jnp.float32)
    W2_all = jnp.dot(XbA_all, kk, preferred_element_type=jnp.float32)

    U_l = []
    for m in range(M):
        S0 = S0_all[m * DK:(m + 1) * DK]
        U_l.append(W1_all[m * C:(m + 1) * C] -
                   jnp.dot(W2_all[m * C:(m + 1) * C], S0,
                           preferred_element_type=jnp.float32))
    U_all = jnp.concatenate(U_l, axis=0)

    o_ref[h, c * C:(c + 1) * C] = \
        jnp.dot(qA_all, S0_all, preferred_element_type=jnp.float32) + \
        jnp.dot(attn_all, U_all, preferred_element_type=jnp.float32)

    kdec = jnp.exp(G_cols[C - 1:C, :] - G_cols)
    aC_row = A_cols[C - 1:C, :]
    Snew_l = []
    for m in range(M):
        S0 = S0_all[m * DK:(m + 1) * DK]
        Snew_l.append(aC_row[0:1, m:m + 1] * S0 + jax.lax.dot_general(
            kdec[:, m:m + 1] * kk, U_l[m], (((0,), (0,)), ((), ())),
            preferred_element_type=jnp.float32))
    state_ref[h] = jnp.concatenate(Snew_l, axis=0)


def _out_kernel(o_ref, gate_ref, onw_ref, wot_ref, out_ref):
    ys = []
    for h in range(NH):
        oh = o_ref[h]
        ms = jnp.sum(oh * oh, axis=1, keepdims=True) * (1.0 / DV)
        rms = oh * jax.lax.rsqrt(ms + EPS)
        ys.append(rms * onw_ref[0:1, :] * gate_ref[:, h * DV:(h + 1) * DV])
    y = jnp.concatenate(ys, axis=1)
    out_ref[...] = jnp.dot(y, wot_ref[...], preferred_element_type=jnp.float32)


@jax.jit
def kernel(hidden_states, Wq, Wk, Wv, Wb, Wa, Wg_router, A_log, dt_bias,
           Wg_gate, o_norm_w, Wo):
    hs = hidden_states.reshape(S, HID)
    wbig = jnp.concatenate([Wq, Wk, Wv, Wg_gate], axis=0).T
    wsmall = jnp.concatenate([Wb, Wa], axis=0)
    adT = jnp.concatenate([A_log, dt_bias]).reshape(2 * NH, 1)

    logitsT = pl.pallas_call(
        _logits_kernel,
        grid=(S // TS,),
        in_specs=[
            pl.BlockSpec((TS, HID), lambda t: (t, 0)),
            pl.BlockSpec((M, HID), lambda t: (0, 0)),
        ],
        out_specs=pl.BlockSpec((M, TS), lambda t: (0, t)),
        out_shape=jax.ShapeDtypeStruct((M, S), jnp.float32),
    )(hs, Wg_router)

    mesh = plsc.VectorSubcoreMesh(core_axis_name="c", subcore_axis_name="s")
    rwT = functools.partial(
        pl.kernel, mesh=mesh,
        out_type=jax.ShapeDtypeStruct((M, S), jnp.float32),
        scratch_types=[
            pltpu.VMEM((M, _TPW), jnp.float32),
            pltpu.VMEM((M, _TPW), jnp.float32),
        ],
    )(_router_sc)(logitsT)

    q, k, v, gate, bg = pl.pallas_call(
        _proj_kernel,
        grid=(S // TS,),
        in_specs=[
            pl.BlockSpec((TS, HID), lambda t: (t, 0)),
            pl.BlockSpec((HID, 3 * KD + VD), lambda t: (0, 0)),
            pl.BlockSpec((2 * NH, HID), lambda t: (0, 0)),
            pl.BlockSpec((2 * NH, 1), lambda t: (0, 0)),
        ],
        out_specs=[
            pl.BlockSpec((NH, TS, DK), lambda t: (0, t, 0)),
            pl.BlockSpec((NH, TS, DK), lambda t: (0, t, 0)),
            pl.BlockSpec((NH, TS, DV), lambda t: (0, t, 0)),
            pl.BlockSpec((TS, VD), lambda t: (t, 0)),
            pl.BlockSpec((2 * NH, TS), lambda t: (0, t)),
        ],
        out_shape=[
            jax.ShapeDtypeStruct((NH, S, DK), jnp.float32),
            jax.ShapeDtypeStruct((NH, S, DK), jnp.float32),
            jax.ShapeDtypeStruct((NH, S, DV), jnp.float32),
            jax.ShapeDtypeStruct((S, VD), jnp.float32),
            jax.ShapeDtypeStruct((2 * NH, S), jnp.float32),
        ],
    )(hs, wbig, wsmall, adT)

    bg_r = bg.reshape(2 * NH, NC, C).transpose(1, 0, 2)
    rw_r = rwT.reshape(M, NC, C).transpose(1, 0, 2)

    o = pl.pallas_call(
        _scan_kernel,
        grid=(NC // 2,),
        in_specs=[
            pl.BlockSpec((NH, 2 * C, DK), lambda n: (0, n, 0)),
            pl.BlockSpec((NH, 2 * C, DK), lambda n: (0, n, 0)),
            pl.BlockSpec((NH, 2 * C, DV), lambda n: (0, n, 0)),
            pl.BlockSpec((2, 2 * NH, C), lambda n: (n, 0, 0)),
            pl.BlockSpec((2, M, C), lambda n: (n, 0, 0)),
        ],
        out_specs=pl.BlockSpec((NH, 2 * C, DV), lambda n: (0, n, 0)),
        out_shape=jax.ShapeDtypeStruct((NH, S, DV), jnp.float32),
        scratch_shapes=[
            pltpu.VMEM((NH, M * DK, DV), jnp.float32),
        ],
        compiler_params=pltpu.CompilerParams(
            dimension_semantics=("arbitrary",)),
    )(k, q, v, bg_r, rw_r)

    out = pl.pallas_call(
        _out_kernel,
        grid=(S // TS,),
        in_specs=[
            pl.BlockSpec((NH, TS, DV), lambda t: (0, t, 0)),
            pl.BlockSpec((TS, VD), lambda t: (t, 0)),
            pl.BlockSpec((1, DV), lambda t: (0, 0)),
            pl.BlockSpec((VD, HID), lambda t: (0, 0)),
        ],
        out_specs=pl.BlockSpec((TS, HID), lambda t: (t, 0)),
        out_shape=jax.ShapeDtypeStruct((S, HID), jnp.float32),
    )(o, gate, o_norm_w.reshape(1, DV), Wo.T)

    return out.reshape(B, S, HID)
